# baseline (device time: 45488 ns/iter reference)
import jax
import jax.numpy as jnp
from jax import lax
from jax.experimental import pallas as pl
from jax.experimental.pallas import tpu as pltpu

N_DEV = 16
E_LOCAL = 4
N_TOK = 1024
D = 512
H = 1024
N_EXP = 64
ROWS = N_TOK // N_DEV


def kernel(x, router_W, route_idx, expert_W, shared_W):
    def body(x_ref, rw_ref, idx_ref, ew_ref, sw_ref, out_ref,
             comm_ref, recv_ref, send_sems, recv_sems):
        my = lax.axis_index("i")

        xf = x_ref[...]
        xb = xf.astype(jnp.bfloat16)

        scores = jnp.dot(xf, rw_ref[...], preferred_element_type=jnp.float32)
        m = jnp.max(scores, axis=-1, keepdims=True)
        p = jnp.exp(scores - m)
        probs = p / jnp.sum(p, axis=-1, keepdims=True)

        ridx = idx_ref[...]
        eids = lax.broadcasted_iota(jnp.int32, (N_TOK, N_EXP), 1)

        acc = jnp.zeros((N_TOK, H), jnp.float32)
        for j in range(E_LOCAL):
            e = my * E_LOCAL + j
            p_e = jnp.sum(jnp.where(eids == e, probs, 0.0),
                          axis=1, keepdims=True)
            gate = jnp.where(ridx == e, p_e, 0.0)
            y = jnp.dot(xb, ew_ref[j].astype(jnp.bfloat16),
                        preferred_element_type=jnp.float32)
            acc = acc + gate * y
        comm_ref[...] = acc.astype(jnp.bfloat16)

        sends = []
        for off in range(N_DEV):
            dst = (my + off) % N_DEV
            rdma = pltpu.make_async_remote_copy(
                src_ref=comm_ref.at[pl.ds(dst * ROWS, ROWS), :],
                dst_ref=recv_ref.at[my],
                send_sem=send_sems.at[off],
                recv_sem=recv_sems.at[my],
                device_id=(dst,),
                device_id_type=pl.DeviceIdType.MESH,
            )
            rdma.start()
            sends.append(rdma)

        x_blk = x_ref[pl.ds(my * ROWS, ROWS), :].astype(jnp.bfloat16)
        shared = jnp.dot(x_blk, sw_ref[...].astype(jnp.bfloat16),
                         preferred_element_type=jnp.float32)

        for src in range(N_DEV):
            recv = pltpu.make_async_remote_copy(
                src_ref=comm_ref.at[pl.ds(0, ROWS), :],
                dst_ref=recv_ref.at[src],
                send_sem=send_sems.at[0],
                recv_sem=recv_sems.at[src],
                device_id=(my,),
                device_id_type=pl.DeviceIdType.MESH,
            )
            recv.wait_recv()

        total = shared
        for src in range(N_DEV):
            total = total + recv_ref[src].astype(jnp.float32)
        out_ref[...] = total

        for rdma in sends:
            rdma.wait_send()

    return pl.pallas_call(
        body,
        out_shape=jax.ShapeDtypeStruct((ROWS, H), jnp.float32),
        in_specs=[pl.BlockSpec(memory_space=pltpu.VMEM)] * 5,
        out_specs=pl.BlockSpec(memory_space=pltpu.VMEM),
        scratch_shapes=[
            pltpu.VMEM((N_TOK, H), jnp.bfloat16),
            pltpu.VMEM((N_DEV, ROWS, H), jnp.bfloat16),
            pltpu.SemaphoreType.DMA((N_DEV,)),
            pltpu.SemaphoreType.DMA((N_DEV,)),
        ],
    )(x, router_W, route_idx, expert_W, shared_W)


# device time: 42141 ns/iter; 1.0794x vs baseline; 1.0794x over previous
import jax
import jax.numpy as jnp
from jax import lax
from jax.experimental import pallas as pl
from jax.experimental.pallas import tpu as pltpu

N_DEV = 16
E_LOCAL = 4
N_TOK = 1024
D = 512
H = 1024
N_EXP = 64
ROWS = N_TOK // N_DEV
CHUNK = 128
N_CHUNK = N_TOK // CHUNK
D_PER_CHUNK = CHUNK // ROWS


def kernel(x, router_W, route_idx, expert_W, shared_W):
    def body(x_ref, rw_ref, idx_ref, ew_ref, sw_ref, out_ref,
             comm_ref, recv_ref, send_sems, recv_sems):
        my = lax.axis_index("i")

        xf = x_ref[...]
        xb = xf.astype(jnp.bfloat16)
        ewb = ew_ref[...].astype(jnp.bfloat16)

        scores = jnp.dot(xf, rw_ref[...], preferred_element_type=jnp.float32)
        m = jnp.max(scores, axis=-1, keepdims=True)
        p = jnp.exp(scores - m)
        probs = p / jnp.sum(p, axis=-1, keepdims=True)

        ridx = idx_ref[...]
        eids = lax.broadcasted_iota(jnp.int32, (CHUNK, N_EXP), 1)

        sends = []
        for b in range(N_CHUNK):
            r0 = b * CHUNK
            xc = xb[r0:r0 + CHUNK]
            pc = probs[r0:r0 + CHUNK]
            rc = ridx[r0:r0 + CHUNK]
            acc = jnp.zeros((CHUNK, H), jnp.float32)
            for j in range(E_LOCAL):
                e = my * E_LOCAL + j
                p_e = jnp.sum(jnp.where(eids == e, pc, 0.0),
                              axis=1, keepdims=True)
                gate = jnp.where(rc == e, p_e, 0.0)
                acc = acc + gate * jnp.dot(
                    xc, ewb[j], preferred_element_type=jnp.float32)
            comm_ref[pl.ds(r0, CHUNK), :] = acc.astype(jnp.bfloat16)
            for k in range(D_PER_CHUNK):
                dst = b * D_PER_CHUNK + k
                rdma = pltpu.make_async_remote_copy(
                    src_ref=comm_ref.at[pl.ds(dst * ROWS, ROWS), :],
                    dst_ref=recv_ref.at[my],
                    send_sem=send_sems.at[dst],
                    recv_sem=recv_sems.at[my],
                    device_id=(dst,),
                    device_id_type=pl.DeviceIdType.MESH,
                )
                rdma.start()
                sends.append(rdma)

        x_blk = x_ref[pl.ds(my * ROWS, ROWS), :].astype(jnp.bfloat16)
        shared = jnp.dot(x_blk, sw_ref[...].astype(jnp.bfloat16),
                         preferred_element_type=jnp.float32)

        for src in range(N_DEV):
            recv = pltpu.make_async_remote_copy(
                src_ref=comm_ref.at[pl.ds(0, ROWS), :],
                dst_ref=recv_ref.at[src],
                send_sem=send_sems.at[0],
                recv_sem=recv_sems.at[src],
                device_id=(my,),
                device_id_type=pl.DeviceIdType.MESH,
            )
            recv.wait_recv()

        total = shared
        for src in range(N_DEV):
            total = total + recv_ref[src].astype(jnp.float32)
        out_ref[...] = total

        for rdma in sends:
            rdma.wait_send()

    return pl.pallas_call(
        body,
        out_shape=jax.ShapeDtypeStruct((ROWS, H), jnp.float32),
        in_specs=[pl.BlockSpec(memory_space=pltpu.VMEM)] * 5,
        out_specs=pl.BlockSpec(memory_space=pltpu.VMEM),
        scratch_shapes=[
            pltpu.VMEM((N_TOK, H), jnp.bfloat16),
            pltpu.VMEM((N_DEV, ROWS, H), jnp.bfloat16),
            pltpu.SemaphoreType.DMA((N_DEV,)),
            pltpu.SemaphoreType.DMA((N_DEV,)),
        ],
    )(x, router_W, route_idx, expert_W, shared_W)


# device time: 32754 ns/iter; 1.3888x vs baseline; 1.2866x over previous
import jax
import jax.numpy as jnp
from jax import lax
from jax.experimental import pallas as pl
from jax.experimental.pallas import tpu as pltpu

N_DEV = 16
E_LOCAL = 4
N_TOK = 1024
D = 512
H = 1024
N_EXP = 64
ROWS = N_TOK // N_DEV
MY_CAP = 128
PAIR_CAP = 24
MSG_W = H + 128

F32 = jnp.float32
BF16 = jnp.bfloat16


def kernel(x, router_W, route_idx, expert_W, shared_W):
    route_row = route_idx.reshape(1, N_TOK)

    def body(x_ref, rw_ref, idx_ref, rrow_ref, ew_ref, sw_ref, out_ref,
             send_ref, recv_ref, yext_ref, send_sems, recv_sems):
        my = lax.axis_index("i")

        xf = x_ref[...]
        xb = xf.astype(BF16)
        ewb = ew_ref[...].astype(BF16)

        scores = jnp.dot(xf, rw_ref[...], preferred_element_type=F32)
        m = jnp.max(scores, axis=-1, keepdims=True)
        p = jnp.exp(scores - m)
        probs = p / jnp.sum(p, axis=-1, keepdims=True)

        ridx_c = idx_ref[...]
        ridx_r = rrow_ref[...]
        e_lo = my * E_LOCAL
        e_hi = e_lo + E_LOCAL

        eids = lax.broadcasted_iota(jnp.int32, (N_TOK, N_EXP), 1)
        gate_c = jnp.zeros((N_TOK, 1), F32)
        for j in range(E_LOCAL):
            e = e_lo + j
            p_e = jnp.sum(jnp.where(eids == e, probs, 0.0),
                          axis=1, keepdims=True)
            gate_c = gate_c + jnp.where(ridx_c == e, p_e, 0.0)

        ti_c = lax.broadcasted_iota(jnp.int32, (N_TOK, 1), 0)
        ti_r = lax.broadcasted_iota(jnp.int32, (1, N_TOK), 1)
        lt = (ti_c <= ti_r).astype(BF16)
        mine_r = ((ridx_r >= e_lo) & (ridx_r < e_hi))
        mine_c = ((ridx_c >= e_lo) & (ridx_c < e_hi))
        lt_ge = (ti_c >= ti_r).astype(BF16)
        pos_r = jnp.dot(mine_r.astype(BF16), lt, preferred_element_type=F32)
        pos_c = jnp.dot(lt_ge, mine_c.astype(BF16),
                        preferred_element_type=F32)

        rk_c = lax.broadcasted_iota(jnp.int32, (MY_CAP, 1), 0).astype(F32)
        rk_r = lax.broadcasted_iota(jnp.int32, (1, MY_CAP), 1).astype(F32)
        g = ((pos_r == rk_c + 1.0) & mine_r).astype(F32)
        gt = ((pos_c == rk_r + 1.0) & mine_c).astype(F32)

        lid_tok_c = (ti_c % ROWS).astype(F32)
        dhi_tok_r = (ti_r // ROWS).astype(F32)
        lidx = jnp.dot(g, lid_tok_c, preferred_element_type=F32)
        dhi_r = jnp.dot(dhi_tok_r, gt, preferred_element_type=F32)
        val_r = jnp.dot(jnp.ones((1, N_TOK), F32), gt,
                        preferred_element_type=F32)
        gv = jnp.dot(g, gate_c, preferred_element_type=F32)
        etok = jnp.dot(g, ridx_c.astype(F32),
                       preferred_element_type=F32)
        xg = jnp.dot(g.astype(BF16), xb,
                     preferred_element_type=F32).astype(BF16)

        y = jnp.zeros((MY_CAP, H), F32)
        for j in range(E_LOCAL):
            ym = jnp.dot(xg, ewb[j], preferred_element_type=F32)
            y = y + jnp.where(etok == (e_lo + j).astype(F32), ym, 0.0)
        y = gv * y

        yext_ref[:, 0:H] = y.astype(BF16)
        yext_ref[:, H:H + 1] = lidx.astype(BF16)
        yext_ref[:, H + 1:] = jnp.zeros((MY_CAP, MSG_W - H - 1), BF16)
        yext = yext_ref[...]

        pr_c = lax.broadcasted_iota(jnp.int32, (PAIR_CAP, 1), 0).astype(F32)
        mi_c = lax.broadcasted_iota(jnp.int32, (MY_CAP, 1), 0)
        mi_r = lax.broadcasted_iota(jnp.int32, (1, MY_CAP), 1)
        lt128 = (mi_c <= mi_r).astype(BF16)
        sends = []
        for d in range(N_DEV):
            md_r = ((dhi_r == float(d)) & (val_r > 0.5))
            posd = jnp.dot(md_r.astype(BF16), lt128,
                           preferred_element_type=F32)
            sel = ((posd == pr_c + 1.0) & md_r).astype(BF16)
            send_ref[d] = jnp.dot(sel, yext, preferred_element_type=F32
                                  ).astype(BF16)
            rdma = pltpu.make_async_remote_copy(
                src_ref=send_ref.at[d],
                dst_ref=recv_ref.at[my],
                send_sem=send_sems.at[d],
                recv_sem=recv_sems.at[my],
                device_id=(d,),
                device_id_type=pl.DeviceIdType.MESH,
            )
            rdma.start()
            sends.append(rdma)

        x_blk = x_ref[pl.ds(my * ROWS, ROWS), :].astype(BF16)
        total = jnp.dot(x_blk, sw_ref[...].astype(BF16),
                        preferred_element_type=F32)

        oi_r = lax.broadcasted_iota(jnp.int32, (1, ROWS), 1).astype(F32)
        for src in range(N_DEV):
            recv = pltpu.make_async_remote_copy(
                src_ref=send_ref.at[src],
                dst_ref=recv_ref.at[src],
                send_sem=send_sems.at[0],
                recv_sem=recv_sems.at[src],
                device_id=(my,),
                device_id_type=pl.DeviceIdType.MESH,
            )
            recv.wait_recv()
        for src in range(N_DEV):
            blk = recv_ref[src]
            idx_c = blk[:, H:H + 1].astype(F32)
            st = (idx_c == oi_r).astype(BF16)
            total = total + lax.dot_general(
                st, blk[:, 0:H],
                dimension_numbers=(((0,), (0,)), ((), ())),
                preferred_element_type=F32)
        out_ref[...] = total

        for rdma in sends:
            rdma.wait_send()

    return pl.pallas_call(
        body,
        out_shape=jax.ShapeDtypeStruct((ROWS, H), F32),
        in_specs=[pl.BlockSpec(memory_space=pltpu.VMEM)] * 6,
        out_specs=pl.BlockSpec(memory_space=pltpu.VMEM),
        scratch_shapes=[
            pltpu.VMEM((N_DEV, PAIR_CAP, MSG_W), BF16),
            pltpu.VMEM((N_DEV, PAIR_CAP, MSG_W), BF16),
            pltpu.VMEM((MY_CAP, MSG_W), BF16),
            pltpu.SemaphoreType.DMA((N_DEV,)),
            pltpu.SemaphoreType.DMA((N_DEV,)),
        ],
    )(x, router_W, route_idx, route_row, expert_W, shared_W)


# device time: 29237 ns/iter; 1.5558x vs baseline; 1.1203x over previous
import jax
import jax.numpy as jnp
from jax import lax
from jax.experimental import pallas as pl
from jax.experimental.pallas import tpu as pltpu

N_DEV = 16
E_LOCAL = 4
N_TOK = 1024
D = 512
H = 1024
N_EXP = 64
ROWS = N_TOK // N_DEV
MY_CAP = 128
PAIR_CAP = 16
MSG_W = H + 128

F32 = jnp.float32
BF16 = jnp.bfloat16


def _dot_t(a, b):
    return lax.dot_general(a, b, dimension_numbers=(((0,), (0,)), ((), ())),
                           preferred_element_type=F32)


def kernel(x, router_W, route_idx, expert_W, shared_W):
    def body(x_ref, rw_ref, idx_ref, ew_ref, sw_ref, out_ref,
             send_ref, recv_ref, yext_ref, send_sems, recv_sems):
        my = lax.axis_index("i")
        xf = x_ref[...]

        scores = jnp.dot(xf, rw_ref[...], preferred_element_type=F32)
        m = jnp.max(scores, axis=-1, keepdims=True)
        p = jnp.exp(scores - m)
        probs = p / jnp.sum(p, axis=-1, keepdims=True)

        ridx_c = idx_ref[...]
        e_lo = my * E_LOCAL
        e_hi = e_lo + E_LOCAL

        eids = lax.broadcasted_iota(jnp.int32, (N_TOK, N_EXP), 1)
        gate_c = jnp.zeros((N_TOK, 1), F32)
        for j in range(E_LOCAL):
            e = e_lo + j
            p_e = jnp.sum(jnp.where(eids == e, probs, 0.0),
                          axis=1, keepdims=True)
            gate_c = gate_c + jnp.where(ridx_c == e, p_e, 0.0)

        ti_c = lax.broadcasted_iota(jnp.int32, (N_TOK, 1), 0)
        ti_r = lax.broadcasted_iota(jnp.int32, (1, N_TOK), 1)
        lt_ge = (ti_c >= ti_r).astype(BF16)
        mine_c = ((ridx_c >= e_lo) & (ridx_c < e_hi))
        pos_c = jnp.dot(lt_ge, mine_c.astype(BF16),
                        preferred_element_type=F32)
        rk_r = lax.broadcasted_iota(jnp.int32, (1, MY_CAP), 1).astype(F32)
        gt = ((pos_c == rk_r + 1.0) & mine_c).astype(F32)

        xg = _dot_t(gt, xf)
        lidx = _dot_t(gt, (ti_c % ROWS).astype(F32))
        gv = _dot_t(gt, gate_c)
        etok = _dot_t(gt, ridx_c.astype(F32))
        dhi_r = jnp.dot((ti_r // ROWS).astype(F32), gt,
                        preferred_element_type=F32)
        val_r = jnp.dot(jnp.ones((1, N_TOK), F32), gt,
                        preferred_element_type=F32)

        y = jnp.zeros((MY_CAP, H), F32)
        for j in range(E_LOCAL):
            ym = jnp.dot(xg, ew_ref[j], preferred_element_type=F32)
            y = y + jnp.where(etok == (e_lo + j).astype(F32), ym, 0.0)
        y = gv * y

        yext_ref[:, 0:H] = y.astype(BF16)
        yext_ref[:, H:H + 1] = lidx.astype(BF16)
        yext_ref[:, H + 1:] = jnp.zeros((MY_CAP, MSG_W - H - 1), BF16)

        di_c = lax.broadcasted_iota(jnp.int32, (N_DEV, 1), 0).astype(F32)
        mi_c = lax.broadcasted_iota(jnp.int32, (MY_CAP, 1), 0)
        mi_r = lax.broadcasted_iota(jnp.int32, (1, MY_CAP), 1)
        lt128 = (mi_c <= mi_r).astype(BF16)
        md = ((dhi_r == di_c) & (val_r > 0.5)).astype(BF16)
        posd = jnp.dot(md, lt128, preferred_element_type=F32)
        big_i = lax.broadcasted_iota(jnp.int32, (N_DEV * PAIR_CAP, 1), 0)
        oh16 = ((big_i // PAIR_CAP) ==
                lax.broadcasted_iota(jnp.int32, (1, N_DEV), 1)).astype(BF16)
        posd_big = jnp.dot(oh16, posd.astype(BF16),
                           preferred_element_type=F32)
        md_big = jnp.dot(oh16, md, preferred_element_type=F32)
        r_big = (big_i % PAIR_CAP).astype(F32)
        sel = ((posd_big == r_big + 1.0) & (md_big > 0.5)).astype(BF16)
        msgs = jnp.dot(sel, yext_ref[...],
                       preferred_element_type=F32)
        send_ref[...] = msgs.astype(BF16).reshape(N_DEV, PAIR_CAP, MSG_W)

        sends = []
        for d in range(N_DEV):
            rdma = pltpu.make_async_remote_copy(
                src_ref=send_ref.at[d],
                dst_ref=recv_ref.at[my],
                send_sem=send_sems.at[d],
                recv_sem=recv_sems.at[my],
                device_id=(d,),
                device_id_type=pl.DeviceIdType.MESH,
            )
            rdma.start()
            sends.append(rdma)

        x_blk = x_ref[pl.ds(my * ROWS, ROWS), :]
        total = jnp.dot(x_blk, sw_ref[...],
                        preferred_element_type=F32)

        for src in range(N_DEV):
            recv = pltpu.make_async_remote_copy(
                src_ref=send_ref.at[src],
                dst_ref=recv_ref.at[src],
                send_sem=send_sems.at[0],
                recv_sem=recv_sems.at[src],
                device_id=(my,),
                device_id_type=pl.DeviceIdType.MESH,
            )
            recv.wait_recv()

        r2 = recv_ref[...].reshape(N_DEV * PAIR_CAP, MSG_W)
        oi_r = lax.broadcasted_iota(jnp.int32, (1, ROWS), 1).astype(F32)
        idx_c = r2[:, H:H + 1].astype(F32)
        st = (idx_c == oi_r).astype(BF16)
        out_ref[...] = total + _dot_t(st, r2[:, 0:H])

        for rdma in sends:
            rdma.wait_send()

    return pl.pallas_call(
        body,
        out_shape=jax.ShapeDtypeStruct((ROWS, H), F32),
        in_specs=[pl.BlockSpec(memory_space=pltpu.VMEM)] * 5,
        out_specs=pl.BlockSpec(memory_space=pltpu.VMEM),
        scratch_shapes=[
            pltpu.VMEM((N_DEV, PAIR_CAP, MSG_W), BF16),
            pltpu.VMEM((N_DEV, PAIR_CAP, MSG_W), BF16),
            pltpu.VMEM((MY_CAP, MSG_W), BF16),
            pltpu.SemaphoreType.DMA((N_DEV,)),
            pltpu.SemaphoreType.DMA((N_DEV,)),
        ],
    )(x, router_W, route_idx, expert_W, shared_W)


# device time: 22797 ns/iter; 1.9954x vs baseline; 1.2825x over previous
import jax
import jax.numpy as jnp
from jax import lax
from jax.experimental import pallas as pl
from jax.experimental.pallas import tpu as pltpu

N_DEV = 16
E_LOCAL = 4
N_TOK = 1024
D = 512
H = 1024
N_EXP = 64
ROWS = N_TOK // N_DEV
MY_CAP = 128
PAIR_CAP = 16
MSG_W = H + 128

F32 = jnp.float32
BF16 = jnp.bfloat16


def _dot_t(a, b):
    return lax.dot_general(a, b, dimension_numbers=(((0,), (0,)), ((), ())),
                           preferred_element_type=F32)


def kernel(x, router_W, route_idx, expert_W, shared_W):
    def body(x_ref, rw_ref, idx_ref, ew_ref, sw_ref, out_ref,
             send_ref, recv_ref, yext_ref, send_sems, recv_sems,
             credit_sems):
        my = lax.axis_index("i")

        bsem = pltpu.get_barrier_semaphore()
        pl.semaphore_signal(bsem, inc=1, device_id=(my,),
                            device_id_type=pl.DeviceIdType.MESH)
        pl.semaphore_wait(bsem, 1)
        for d in range(N_DEV):
            pl.semaphore_signal(credit_sems.at[my], inc=1, device_id=(d,),
                                device_id_type=pl.DeviceIdType.MESH)

        xf = x_ref[...]

        scores = jnp.dot(xf, rw_ref[...], preferred_element_type=F32)
        m = jnp.max(scores, axis=-1, keepdims=True)
        p = jnp.exp(scores - m)
        probs = p / jnp.sum(p, axis=-1, keepdims=True)

        ridx_c = idx_ref[...]
        e_lo = my * E_LOCAL
        e_hi = e_lo + E_LOCAL

        eids = lax.broadcasted_iota(jnp.int32, (N_TOK, N_EXP), 1)
        gate_c = jnp.zeros((N_TOK, 1), F32)
        for j in range(E_LOCAL):
            e = e_lo + j
            p_e = jnp.sum(jnp.where(eids == e, probs, 0.0),
                          axis=1, keepdims=True)
            gate_c = gate_c + jnp.where(ridx_c == e, p_e, 0.0)

        ti_c = lax.broadcasted_iota(jnp.int32, (N_TOK, 1), 0)
        ti_r = lax.broadcasted_iota(jnp.int32, (1, N_TOK), 1)
        lt_ge = (ti_c >= ti_r).astype(BF16)
        mine_c = ((ridx_c >= e_lo) & (ridx_c < e_hi))
        pos_c = jnp.dot(lt_ge, mine_c.astype(BF16),
                        preferred_element_type=F32)
        rk_r = lax.broadcasted_iota(jnp.int32, (1, MY_CAP), 1).astype(F32)
        gt = ((pos_c == rk_r + 1.0) & mine_c).astype(F32)

        xg = _dot_t(gt, xf)
        lidx = _dot_t(gt, (ti_c % ROWS).astype(F32))
        gv = _dot_t(gt, gate_c)
        etok = _dot_t(gt, ridx_c.astype(F32))
        dhi_r = jnp.dot((ti_r // ROWS).astype(F32), gt,
                        preferred_element_type=F32)
        val_r = jnp.dot(jnp.ones((1, N_TOK), F32), gt,
                        preferred_element_type=F32)

        y = jnp.zeros((MY_CAP, H), F32)
        for j in range(E_LOCAL):
            ym = jnp.dot(xg, ew_ref[j], preferred_element_type=F32)
            y = y + jnp.where(etok == (e_lo + j).astype(F32), ym, 0.0)
        y = gv * y

        yext_ref[:, 0:H] = y.astype(BF16)
        yext_ref[:, H:H + 1] = lidx.astype(BF16)
        yext_ref[:, H + 1:] = jnp.zeros((MY_CAP, MSG_W - H - 1), BF16)

        di_c = lax.broadcasted_iota(jnp.int32, (N_DEV, 1), 0).astype(F32)
        mi_c = lax.broadcasted_iota(jnp.int32, (MY_CAP, 1), 0)
        mi_r = lax.broadcasted_iota(jnp.int32, (1, MY_CAP), 1)
        lt128 = (mi_c <= mi_r).astype(BF16)
        md = ((dhi_r == di_c) & (val_r > 0.5)).astype(BF16)
        posd = jnp.dot(md, lt128, preferred_element_type=F32)
        big_i = lax.broadcasted_iota(jnp.int32, (N_DEV * PAIR_CAP, 1), 0)
        oh16 = ((big_i // PAIR_CAP) ==
                lax.broadcasted_iota(jnp.int32, (1, N_DEV), 1)).astype(BF16)
        posd_big = jnp.dot(oh16, posd.astype(BF16),
                           preferred_element_type=F32)
        md_big = jnp.dot(oh16, md, preferred_element_type=F32)
        r_big = (big_i % PAIR_CAP).astype(F32)
        sel = ((posd_big == r_big + 1.0) & (md_big > 0.5)).astype(BF16)
        msgs = jnp.dot(sel, yext_ref[...],
                       preferred_element_type=F32)
        send_ref[...] = msgs.astype(BF16).reshape(N_DEV, PAIR_CAP, MSG_W)

        sends = []
        for d in range(N_DEV):
            pl.semaphore_wait(credit_sems.at[d], 1)
            rdma = pltpu.make_async_remote_copy(
                src_ref=send_ref.at[d],
                dst_ref=recv_ref.at[my],
                send_sem=send_sems.at[d],
                recv_sem=recv_sems.at[my],
                device_id=(d,),
                device_id_type=pl.DeviceIdType.MESH,
            )
            rdma.start()
            sends.append(rdma)

        x_blk = x_ref[pl.ds(my * ROWS, ROWS), :]
        total = jnp.dot(x_blk, sw_ref[...],
                        preferred_element_type=F32)

        for src in range(N_DEV):
            recv = pltpu.make_async_remote_copy(
                src_ref=send_ref.at[src],
                dst_ref=recv_ref.at[src],
                send_sem=send_sems.at[0],
                recv_sem=recv_sems.at[src],
                device_id=(my,),
                device_id_type=pl.DeviceIdType.MESH,
            )
            recv.wait_recv()

        r2 = recv_ref[...].reshape(N_DEV * PAIR_CAP, MSG_W)
        oi_r = lax.broadcasted_iota(jnp.int32, (1, ROWS), 1).astype(F32)
        idx_c = r2[:, H:H + 1].astype(F32)
        st = (idx_c == oi_r).astype(BF16)
        out_ref[...] = total + _dot_t(st, r2[:, 0:H])

        for rdma in sends:
            rdma.wait_send()

    return pl.pallas_call(
        body,
        out_shape=jax.ShapeDtypeStruct((ROWS, H), F32),
        in_specs=[pl.BlockSpec(memory_space=pltpu.VMEM)] * 5,
        out_specs=pl.BlockSpec(memory_space=pltpu.VMEM),
        scratch_shapes=[
            pltpu.VMEM((N_DEV, PAIR_CAP, MSG_W), BF16),
            pltpu.VMEM((N_DEV, PAIR_CAP, MSG_W), BF16),
            pltpu.VMEM((MY_CAP, MSG_W), BF16),
            pltpu.SemaphoreType.DMA((N_DEV,)),
            pltpu.SemaphoreType.DMA((N_DEV,)),
            pltpu.SemaphoreType.REGULAR((N_DEV,)),
        ],
        compiler_params=pltpu.CompilerParams(collective_id=0),
    )(x, router_W, route_idx, expert_W, shared_W)
